# baseline (speedup 1.0000x reference)
import functools

import jax
import jax.numpy as jnp
from jax import lax
from jax.experimental import pallas as pl
from jax.experimental.pallas import tpu as pltpu
from jax.experimental.pallas import tpu_sc as plsc

T = 2048
D = 768
F = 2048
NE = 64
K = 2
TM = 128
MAX_TILES = (T * K) // TM + NE
R = MAX_TILES * TM

NC = 2
NS = 16
NW = NC * NS


def _router_body(x_ref, wg_ref, cnt_ref, aux_ref, w1_ref, w2_ref, p1_ref,
                 p2_ref, eid_ref, val_ref):
    x = x_ref[...]
    logits = jnp.dot(x, wg_ref[...], preferred_element_type=jnp.float32)
    m = jnp.max(logits, axis=-1, keepdims=True)
    p = jnp.exp(logits - m)
    p = p / jnp.sum(p, axis=-1, keepdims=True)
    iota = lax.broadcasted_iota(jnp.int32, (T, NE), 1)
    p1 = jnp.max(p, axis=-1, keepdims=True)
    e1 = jnp.min(jnp.where(p == p1, iota, NE), axis=-1, keepdims=True)
    pm = jnp.where(iota == e1, -1e30, p)
    p2 = jnp.max(pm, axis=-1, keepdims=True)
    e2 = jnp.min(jnp.where(pm == p2, iota, NE), axis=-1, keepdims=True)
    s = p1 + p2
    w1_ref[...] = p1 / s
    w2_ref[...] = p2 / s
    hit1 = (iota == e1).astype(jnp.float32)
    hit2 = (iota == e2).astype(jnp.float32)
    hits = hit1 + hit2
    cnt = jnp.sum(hits, axis=0)
    cnt_ref[...] = cnt
    pmean = jnp.mean(p, axis=0)
    f = cnt / jnp.sum(cnt)
    aux_ref[...] = (NE * jnp.sum(f * pmean))[None, None]

    cnt_row = cnt[None, :]
    tiles_pe = (cnt_row.astype(jnp.int32) + TM - 1) // TM
    i64r = lax.broadcasted_iota(jnp.int32, (NE, NE), 0)
    i64c = lax.broadcasted_iota(jnp.int32, (NE, NE), 1)
    m_le = (i64r <= i64c).astype(jnp.float32)
    tile_cum = jnp.dot(tiles_pe.astype(jnp.float32), m_le,
                       preferred_element_type=jnp.float32).astype(jnp.int32)
    padded_start = ((tile_cum - tiles_pe) * TM).astype(jnp.float32)

    tb = 256
    ibr = lax.broadcasted_iota(jnp.int32, (tb, tb), 0)
    ibc = lax.broadcasted_iota(jnp.int32, (tb, tb), 1)
    l_strict = (ibc < ibr).astype(jnp.float32)
    off = jnp.zeros((1, NE), jnp.float32)
    for b in range(T // tb):
        sl = slice(b * tb, (b + 1) * tb)
        hb = hits[sl]
        cumex_b = jnp.dot(l_strict, hb,
                          preferred_element_type=jnp.float32) + off
        off = off + jnp.sum(hb, axis=0, keepdims=True)
        tgt_b = padded_start + cumex_b
        p1_ref[sl, :] = jnp.sum(hit1[sl] * tgt_b, axis=-1,
                                keepdims=True).astype(jnp.int32)
        p2_ref[sl, :] = jnp.sum(hit2[sl] * tgt_b, axis=-1,
                                keepdims=True).astype(jnp.int32)

    ti = lax.broadcasted_iota(jnp.int32, (MAX_TILES, NE), 0)
    eid = jnp.sum((tile_cum <= ti).astype(jnp.int32), axis=-1,
                  keepdims=True)
    eid_ref[...] = jnp.minimum(eid, NE - 1)
    total = jnp.max(tile_cum)
    vi = lax.broadcasted_iota(jnp.int32, (MAX_TILES, 1), 0)
    val_ref[...] = (vi < total).astype(jnp.int32)


def _run_router(x2d, wg):
    return pl.pallas_call(
        _router_body,
        out_shape=(
            jax.ShapeDtypeStruct((NE,), jnp.float32),
            jax.ShapeDtypeStruct((1, 1), jnp.float32),
            jax.ShapeDtypeStruct((T, 1), jnp.float32),
            jax.ShapeDtypeStruct((T, 1), jnp.float32),
            jax.ShapeDtypeStruct((T, 1), jnp.int32),
            jax.ShapeDtypeStruct((T, 1), jnp.int32),
            jax.ShapeDtypeStruct((MAX_TILES, 1), jnp.int32),
            jax.ShapeDtypeStruct((MAX_TILES, 1), jnp.int32),
        ),
    )(x2d, wg)


_C_PER_W = T // NW
_C_VCH = D // 16


@functools.lru_cache(maxsize=None)
def _sc_kernels():
    mesh = plsc.VectorSubcoreMesh(core_axis_name="c", subcore_axis_name="s")

    @functools.partial(
        pl.kernel,
        out_type=jax.ShapeDtypeStruct((T, D), jnp.float32),
        mesh=mesh,
        scratch_types=[
            pltpu.VMEM((_C_PER_W,), jnp.int32),
            pltpu.VMEM((_C_PER_W,), jnp.int32),
            pltpu.VMEM((_C_PER_W, D), jnp.float32),
            pltpu.VMEM((_C_PER_W, D), jnp.float32),
            pltpu.SemaphoreType.DMA,
        ],
    )
    def _sc_combine(ys_hbm, pos0_hbm, pos1_hbm, out_hbm, i0_v, i1_v, b0_v,
                    b1_v, sem):
        wid = lax.axis_index("s") * NC + lax.axis_index("c")
        base = wid * _C_PER_W
        pltpu.sync_copy(pos0_hbm.at[pl.ds(base, _C_PER_W)], i0_v)
        pltpu.sync_copy(pos1_hbm.at[pl.ds(base, _C_PER_W)], i1_v)
        pltpu.async_copy(ys_hbm.at[i0_v], b0_v, sem).wait()
        pltpu.async_copy(ys_hbm.at[i1_v], b1_v, sem).wait()

        def _row(r, carry):
            for c in range(_C_VCH):
                sl = pl.ds(c * 16, 16)
                b0_v[r, sl] = b0_v[r, sl] + b1_v[r, sl]
            return carry

        lax.fori_loop(0, _C_PER_W, _row, 0)
        pltpu.sync_copy(b0_v, out_hbm.at[pl.ds(base, _C_PER_W)])

    return _sc_combine


def _ffn_body(eid_ref, val_ref, x_ref, wg_ref, wu_ref, wd_ref, pos0_ref,
              pos1_ref, w1_ref, w2_ref, out_ref):
    i = pl.program_id(0)

    @pl.when(val_ref[i] == 1)
    def _compute():
        rid = lax.broadcasted_iota(jnp.int32, (TM, T), 0) + i * TM
        m0 = pos0_ref[...] == rid
        m1 = pos1_ref[...] == rid
        sel = (m0 | m1).astype(jnp.float32)
        wvec = jnp.sum(jnp.where(m0, w1_ref[...], 0.0) +
                       jnp.where(m1, w2_ref[...], 0.0),
                       axis=-1, keepdims=True)
        xt = jnp.dot(sel, x_ref[...], preferred_element_type=jnp.float32)
        g = jnp.dot(xt, wg_ref[0], preferred_element_type=jnp.float32)
        u = jnp.dot(xt, wu_ref[0], preferred_element_type=jnp.float32)
        h = g * jax.nn.sigmoid(g) * u
        y = jnp.dot(h, wd_ref[0], preferred_element_type=jnp.float32)
        out_ref[...] = y * wvec


def _run_ffn(tile_eid, tile_valid, x2d, w_gate, w_up, w_down, pos0r, pos1r,
             w1r, w2r):
    grid_spec = pltpu.PrefetchScalarGridSpec(
        num_scalar_prefetch=2,
        grid=(MAX_TILES,),
        in_specs=[
            pl.BlockSpec((T, D), lambda i, eid, val: (0, 0)),
            pl.BlockSpec((1, D, F), lambda i, eid, val: (eid[i], 0, 0)),
            pl.BlockSpec((1, D, F), lambda i, eid, val: (eid[i], 0, 0)),
            pl.BlockSpec((1, F, D), lambda i, eid, val: (eid[i], 0, 0)),
            pl.BlockSpec((1, T), lambda i, eid, val: (0, 0)),
            pl.BlockSpec((1, T), lambda i, eid, val: (0, 0)),
            pl.BlockSpec((1, T), lambda i, eid, val: (0, 0)),
            pl.BlockSpec((1, T), lambda i, eid, val: (0, 0)),
        ],
        out_specs=pl.BlockSpec((TM, D), lambda i, eid, val: (i, 0)),
    )
    return pl.pallas_call(
        _ffn_body,
        grid_spec=grid_spec,
        out_shape=jax.ShapeDtypeStruct((R, D), jnp.float32),
    )(tile_eid, tile_valid, x2d, w_gate, w_up, w_down, pos0r, pos1r, w1r,
      w2r)


def kernel(x, Wg, W_gate, W_up, W_down):
    x2d = x.reshape(T, D)
    cnt_f, aux, w1, w2, pos0, pos1, eid2, val2 = _run_router(x2d, Wg)

    sc_combine = _sc_kernels()
    ys = _run_ffn(eid2.reshape(-1), val2.reshape(-1), x2d, W_gate, W_up,
                  W_down, pos0.reshape(1, T), pos1.reshape(1, T),
                  w1.reshape(1, T), w2.reshape(1, T))
    out2d = sc_combine(ys, pos0.reshape(-1), pos1.reshape(-1))

    return out2d.reshape(1, T, D), aux[0, 0], cnt_f
